# Initial kernel scaffold; baseline (speedup 1.0000x reference)
import functools

import jax
import jax.numpy as jnp
from jax import lax
from jax.experimental import pallas as pl
from jax.experimental.pallas import tpu as pltpu
from jax.experimental.pallas import tpu_sc as plsc

_N = 10000
_E = 320000
_F_IN = 128
_HID = 64
_C = 16

_NC = 2
_NS = 16
_NW = _NC * _NS
_CHUNK = 125
_ROWS = _E // _CHUNK
_ROWS_W = _ROWS // _NW
_NPAD = 10240
_NPS = _NPAD // _NS

_mesh = plsc.VectorSubcoreMesh(
    core_axis_name="c", subcore_axis_name="s", num_cores=_NC, num_subcores=_NS
)


@functools.partial(
    pl.kernel,
    out_type=[
        jax.ShapeDtypeStruct((_NPAD,), jnp.float32),
        jax.ShapeDtypeStruct((_NPAD,), jnp.float32),
    ],
    mesh=_mesh,
    scratch_types=[
        pltpu.VMEM((_ROWS_W, _CHUNK), jnp.int32),
        pltpu.VMEM((128,), jnp.float32),
        pltpu.VMEM_SHARED((_NPAD,), jnp.float32),
    ],
)
def _sc_degree(col_hbm, z1_hbm, cnt0_hbm, cnt1_hbm, colv, ones_v, acc):
    cid = lax.axis_index("c")
    sid = lax.axis_index("s")
    wid = sid * _NC + cid
    pltpu.sync_copy(z1_hbm.at[pl.ds(sid * _NPS, _NPS)], acc.at[pl.ds(sid * _NPS, _NPS)])
    pltpu.sync_copy(col_hbm.at[pl.ds(wid * _ROWS_W, _ROWS_W)], colv)
    for k in range(8):
        ones_v[pl.ds(k * 16, 16)] = jnp.ones((16,), jnp.float32)
    plsc.subcore_barrier()

    def body(j, carry):
        pltpu.sync_copy(ones_v.at[pl.ds(0, _CHUNK)], acc.at[colv.at[j]], add=True)
        return carry

    lax.fori_loop(0, _ROWS_W, body, 0)
    plsc.subcore_barrier()

    @pl.when(cid == 0)
    def _():
        pltpu.sync_copy(acc.at[pl.ds(sid * _NPS, _NPS)], cnt0_hbm.at[pl.ds(sid * _NPS, _NPS)])

    @pl.when(cid == 1)
    def _():
        pltpu.sync_copy(acc.at[pl.ds(sid * _NPS, _NPS)], cnt1_hbm.at[pl.ds(sid * _NPS, _NPS)])


def _make_sc_scatter(depth):
    @functools.partial(
        pl.kernel,
        out_type=[
            jax.ShapeDtypeStruct((_NPAD, depth), jnp.float32),
            jax.ShapeDtypeStruct((_NPAD, depth), jnp.float32),
        ],
        mesh=_mesh,
        scratch_types=[
            pltpu.VMEM((_ROWS_W, _CHUNK), jnp.int32),
            pltpu.VMEM((_ROWS_W, _CHUNK), jnp.int32),
            pltpu.VMEM((_CHUNK, depth), jnp.float32),
            pltpu.VMEM_SHARED((_NPAD, depth), jnp.float32),
            pltpu.SemaphoreType.DMA,
        ],
    )
    def _sc_scatter(row_hbm, col_hbm, y_hbm, zd_hbm, p0_hbm, p1_hbm,
                    rowv, colv, buf, acc, sem):
        cid = lax.axis_index("c")
        sid = lax.axis_index("s")
        wid = sid * _NC + cid
        pltpu.sync_copy(zd_hbm.at[pl.ds(sid * _NPS, _NPS)],
                        acc.at[pl.ds(sid * _NPS, _NPS)])
        pltpu.sync_copy(row_hbm.at[pl.ds(wid * _ROWS_W, _ROWS_W)], rowv)
        pltpu.sync_copy(col_hbm.at[pl.ds(wid * _ROWS_W, _ROWS_W)], colv)
        plsc.subcore_barrier()

        def body(j, carry):
            pltpu.async_copy(y_hbm.at[rowv.at[j]], buf, sem).wait()
            pltpu.sync_copy(buf, acc.at[colv.at[j]], add=True)
            return carry

        lax.fori_loop(0, _ROWS_W, body, 0)
        plsc.subcore_barrier()

        @pl.when(cid == 0)
        def _():
            pltpu.sync_copy(acc.at[pl.ds(sid * _NPS, _NPS)],
                            p0_hbm.at[pl.ds(sid * _NPS, _NPS)])

        @pl.when(cid == 1)
        def _():
            pltpu.sync_copy(acc.at[pl.ds(sid * _NPS, _NPS)],
                            p1_hbm.at[pl.ds(sid * _NPS, _NPS)])

    return _sc_scatter


_sc_scatter_hid = _make_sc_scatter(_HID)
_sc_scatter_out = _make_sc_scatter(_C)


_R = 500
_G = _N // _R


def _tc1_body(x_ref, w1_ref, c0_ref, c1_ref, y_ref, dinv_ref):
    deg = c0_ref[...] + c1_ref[...] + 1.0
    dinv = lax.rsqrt(deg)
    xw = jnp.dot(x_ref[...], w1_ref[...], preferred_element_type=jnp.float32)
    y_ref[...] = xw * dinv
    dinv_ref[...] = dinv


_tc1 = pl.pallas_call(
    _tc1_body,
    grid=(_G,),
    in_specs=[
        pl.BlockSpec((_R, _F_IN), lambda i: (i, 0)),
        pl.BlockSpec((_F_IN, _HID), lambda i: (0, 0)),
        pl.BlockSpec((_R, 1), lambda i: (i, 0)),
        pl.BlockSpec((_R, 1), lambda i: (i, 0)),
    ],
    out_specs=[
        pl.BlockSpec((_R, _HID), lambda i: (i, 0)),
        pl.BlockSpec((_R, 1), lambda i: (i, 0)),
    ],
    out_shape=[
        jax.ShapeDtypeStruct((_N, _HID), jnp.float32),
        jax.ShapeDtypeStruct((_N, 1), jnp.float32),
    ],
)


def _tc2_body(p0_ref, p1_ref, y1_ref, dinv_ref, b1_ref, w2_ref, z_ref):
    dinv = dinv_ref[...]
    out1 = (p0_ref[...] + p1_ref[...] + y1_ref[...]) * dinv + b1_ref[...]
    h = jnp.maximum(out1, 0.0)
    z_ref[...] = jnp.dot(h, w2_ref[...], preferred_element_type=jnp.float32) * dinv


_tc2 = pl.pallas_call(
    _tc2_body,
    grid=(_G,),
    in_specs=[
        pl.BlockSpec((_R, _HID), lambda i: (i, 0)),
        pl.BlockSpec((_R, _HID), lambda i: (i, 0)),
        pl.BlockSpec((_R, _HID), lambda i: (i, 0)),
        pl.BlockSpec((_R, 1), lambda i: (i, 0)),
        pl.BlockSpec((1, _HID), lambda i: (0, 0)),
        pl.BlockSpec((_HID, _C), lambda i: (0, 0)),
    ],
    out_specs=[pl.BlockSpec((_R, _C), lambda i: (i, 0))],
    out_shape=[jax.ShapeDtypeStruct((_N, _C), jnp.float32)],
)


def _tc3_body(q0_ref, q1_ref, z2_ref, dinv_ref, b2_ref, logp_ref, xout_ref):
    xo = (q0_ref[...] + q1_ref[...] + z2_ref[...]) * dinv_ref[...] + b2_ref[...]
    m = jnp.max(xo, axis=1, keepdims=True)
    t = xo - m
    lse = jnp.log(jnp.sum(jnp.exp(t), axis=1, keepdims=True))
    logp_ref[...] = t - lse
    xout_ref[...] = xo


_tc3 = pl.pallas_call(
    _tc3_body,
    grid=(_G,),
    in_specs=[
        pl.BlockSpec((_R, _C), lambda i: (i, 0)),
        pl.BlockSpec((_R, _C), lambda i: (i, 0)),
        pl.BlockSpec((_R, _C), lambda i: (i, 0)),
        pl.BlockSpec((_R, 1), lambda i: (i, 0)),
        pl.BlockSpec((1, _C), lambda i: (0, 0)),
    ],
    out_specs=[
        pl.BlockSpec((_R, _C), lambda i: (i, 0)),
        pl.BlockSpec((_R, _C), lambda i: (i, 0)),
    ],
    out_shape=[
        jax.ShapeDtypeStruct((_N, _C), jnp.float32),
        jax.ShapeDtypeStruct((_N, _C), jnp.float32),
    ],
)


def _ortho_body(w1_ref, w2_ref, o_ref):
    w1 = w1_ref[...]
    w2 = w2_ref[...]
    g1 = lax.dot_general(w1, w1, (((1,), (1,)), ((), ())),
                         preferred_element_type=jnp.float32)
    g2 = lax.dot_general(w2, w2, (((1,), (1,)), ((), ())),
                         preferred_element_type=jnp.float32)
    i1 = (lax.broadcasted_iota(jnp.int32, (_F_IN, _F_IN), 0)
          == lax.broadcasted_iota(jnp.int32, (_F_IN, _F_IN), 1)).astype(jnp.float32)
    i2 = (lax.broadcasted_iota(jnp.int32, (_HID, _HID), 0)
          == lax.broadcasted_iota(jnp.int32, (_HID, _HID), 1)).astype(jnp.float32)
    s1 = jnp.sum((g1 - i1) ** 2)
    s2 = jnp.sum((g2 - i2) ** 2)
    o_ref[...] = jnp.reshape(jnp.sqrt(s1) + jnp.sqrt(s2), (1, 1))


_ortho = pl.pallas_call(
    _ortho_body,
    out_shape=jax.ShapeDtypeStruct((1, 1), jnp.float32),
)


def kernel(x, edge_index, W1, b1, W2, b2):
    row = edge_index[0].reshape(_ROWS, _CHUNK)
    col = edge_index[1].reshape(_ROWS, _CHUNK)
    z1 = jnp.zeros((_NPAD,), jnp.float32)
    z64 = jnp.zeros((_NPAD, _HID), jnp.float32)
    z16 = jnp.zeros((_NPAD, _C), jnp.float32)

    cnt0, cnt1 = _sc_degree(col, z1)
    y1, dinv = _tc1(x, W1, cnt0.reshape(_NPAD, 1), cnt1.reshape(_NPAD, 1))
    p0, p1 = _sc_scatter_hid(row, col, y1, z64)
    (z2,) = _tc2(p0, p1, y1, dinv, b1.reshape(1, _HID), W2)
    q0, q1 = _sc_scatter_out(row, col, z2, z16)
    logp, xout = _tc3(q0, q1, z2, dinv, b2.reshape(1, _C))
    orto = _ortho(W1, W2)
    return (logp, xout, orto.reshape(()))


# speedup vs baseline: 31.8848x; 31.8848x over previous
import functools

import jax
import jax.numpy as jnp
from jax import lax
from jax.experimental import pallas as pl
from jax.experimental.pallas import tpu as pltpu
from jax.experimental.pallas import tpu_sc as plsc

_N = 10000
_E = 320000
_F_IN = 128
_HID = 64
_C = 16

_NC = 2
_NS = 16
_NW = _NC * _NS
_CHUNK = 125
_ROWS = _E // _CHUNK
_ROWS_W = _ROWS // _NW
_NPAD = 10240
_NPS = _NPAD // _NS

_mesh = plsc.VectorSubcoreMesh(
    core_axis_name="c", subcore_axis_name="s", num_cores=_NC, num_subcores=_NS
)


@functools.partial(
    pl.kernel,
    out_type=[
        jax.ShapeDtypeStruct((_NPAD,), jnp.float32),
        jax.ShapeDtypeStruct((_NPAD,), jnp.float32),
    ],
    mesh=_mesh,
    compiler_params=pltpu.CompilerParams(use_tc_tiling_on_sc=False),
    scratch_types=[
        pltpu.VMEM((_ROWS_W, _CHUNK), jnp.int32),
        pltpu.VMEM((128,), jnp.float32),
        pltpu.VMEM_SHARED((_NPAD,), jnp.float32),
    ],
)
def _sc_degree(col_hbm, z1_hbm, cnt0_hbm, cnt1_hbm, colv, ones_v, acc):
    cid = lax.axis_index("c")
    sid = lax.axis_index("s")
    wid = sid * _NC + cid
    pltpu.sync_copy(z1_hbm.at[pl.ds(sid * _NPS, _NPS)], acc.at[pl.ds(sid * _NPS, _NPS)])
    pltpu.sync_copy(col_hbm.at[pl.ds(wid * _ROWS_W, _ROWS_W)], colv)
    for k in range(8):
        ones_v[pl.ds(k * 16, 16)] = jnp.ones((16,), jnp.float32)
    plsc.subcore_barrier()

    def body(j, carry):
        pltpu.sync_copy(ones_v.at[pl.ds(0, _CHUNK)], acc.at[colv.at[j]], add=True)
        return carry

    lax.fori_loop(0, _ROWS_W, body, 0)
    plsc.subcore_barrier()

    @pl.when(cid == 0)
    def _():
        pltpu.sync_copy(acc.at[pl.ds(sid * _NPS, _NPS)], cnt0_hbm.at[pl.ds(sid * _NPS, _NPS)])

    @pl.when(cid == 1)
    def _():
        pltpu.sync_copy(acc.at[pl.ds(sid * _NPS, _NPS)], cnt1_hbm.at[pl.ds(sid * _NPS, _NPS)])


def _make_sc_scatter(depth):
    @functools.partial(
        pl.kernel,
        out_type=[
            jax.ShapeDtypeStruct((_NPAD, depth), jnp.float32),
            jax.ShapeDtypeStruct((_NPAD, depth), jnp.float32),
        ],
        mesh=_mesh,
        compiler_params=pltpu.CompilerParams(use_tc_tiling_on_sc=False),
        scratch_types=[
            pltpu.VMEM((_ROWS_W, _CHUNK), jnp.int32),
            pltpu.VMEM((_ROWS_W, _CHUNK), jnp.int32),
            pltpu.VMEM((_CHUNK, depth), jnp.float32),
            pltpu.VMEM_SHARED((_NPAD, depth), jnp.float32),
            pltpu.SemaphoreType.DMA,
        ],
    )
    def _sc_scatter(row_hbm, col_hbm, y_hbm, zd_hbm, p0_hbm, p1_hbm,
                    rowv, colv, buf, acc, sem):
        cid = lax.axis_index("c")
        sid = lax.axis_index("s")
        wid = sid * _NC + cid
        pltpu.sync_copy(zd_hbm.at[pl.ds(sid * _NPS, _NPS)],
                        acc.at[pl.ds(sid * _NPS, _NPS)])
        pltpu.sync_copy(row_hbm.at[pl.ds(wid * _ROWS_W, _ROWS_W)], rowv)
        pltpu.sync_copy(col_hbm.at[pl.ds(wid * _ROWS_W, _ROWS_W)], colv)
        plsc.subcore_barrier()

        def body(j, carry):
            pltpu.async_copy(y_hbm.at[rowv.at[j]], buf, sem).wait()
            pltpu.sync_copy(buf, acc.at[colv.at[j]], add=True)
            return carry

        lax.fori_loop(0, _ROWS_W, body, 0)
        plsc.subcore_barrier()

        @pl.when(cid == 0)
        def _():
            pltpu.sync_copy(acc.at[pl.ds(sid * _NPS, _NPS)],
                            p0_hbm.at[pl.ds(sid * _NPS, _NPS)])

        @pl.when(cid == 1)
        def _():
            pltpu.sync_copy(acc.at[pl.ds(sid * _NPS, _NPS)],
                            p1_hbm.at[pl.ds(sid * _NPS, _NPS)])

    return _sc_scatter


_sc_scatter_hid = _make_sc_scatter(_HID)
_sc_scatter_out = _make_sc_scatter(_C)


_R = 1000
_G = _N // _R


def _tc1_body(x_ref, w1_ref, c0_ref, c1_ref, y_ref, dinv_ref):
    deg = c0_ref[...] + c1_ref[...] + 1.0
    dinv = lax.rsqrt(deg)
    xw = jnp.dot(x_ref[...], w1_ref[...], preferred_element_type=jnp.float32)
    y_ref[...] = xw * dinv
    dinv_ref[...] = dinv


_tc1 = pl.pallas_call(
    _tc1_body,
    grid=(_G,),
    in_specs=[
        pl.BlockSpec((_R, _F_IN), lambda i: (i, 0)),
        pl.BlockSpec((_F_IN, _HID), lambda i: (0, 0)),
        pl.BlockSpec((_R, 1), lambda i: (i, 0)),
        pl.BlockSpec((_R, 1), lambda i: (i, 0)),
    ],
    out_specs=[
        pl.BlockSpec((_R, _HID), lambda i: (i, 0)),
        pl.BlockSpec((_R, 1), lambda i: (i, 0)),
    ],
    out_shape=[
        jax.ShapeDtypeStruct((_N, _HID), jnp.float32),
        jax.ShapeDtypeStruct((_N, 1), jnp.float32),
    ],
)


def _tc2_body(p0_ref, p1_ref, y1_ref, dinv_ref, b1_ref, w2_ref, z_ref):
    dinv = dinv_ref[...]
    out1 = (p0_ref[...] + p1_ref[...] + y1_ref[...]) * dinv + b1_ref[...]
    h = jnp.maximum(out1, 0.0)
    z_ref[...] = jnp.dot(h, w2_ref[...], preferred_element_type=jnp.float32) * dinv


_tc2 = pl.pallas_call(
    _tc2_body,
    grid=(_G,),
    in_specs=[
        pl.BlockSpec((_R, _HID), lambda i: (i, 0)),
        pl.BlockSpec((_R, _HID), lambda i: (i, 0)),
        pl.BlockSpec((_R, _HID), lambda i: (i, 0)),
        pl.BlockSpec((_R, 1), lambda i: (i, 0)),
        pl.BlockSpec((1, _HID), lambda i: (0, 0)),
        pl.BlockSpec((_HID, _C), lambda i: (0, 0)),
    ],
    out_specs=[pl.BlockSpec((_R, _C), lambda i: (i, 0))],
    out_shape=[jax.ShapeDtypeStruct((_N, _C), jnp.float32)],
)


def _tc3_body(q0_ref, q1_ref, z2_ref, dinv_ref, b2_ref, logp_ref, xout_ref):
    xo = (q0_ref[...] + q1_ref[...] + z2_ref[...]) * dinv_ref[...] + b2_ref[...]
    m = jnp.max(xo, axis=1, keepdims=True)
    t = xo - m
    lse = jnp.log(jnp.sum(jnp.exp(t), axis=1, keepdims=True))
    logp_ref[...] = t - lse
    xout_ref[...] = xo


_tc3 = pl.pallas_call(
    _tc3_body,
    grid=(_G,),
    in_specs=[
        pl.BlockSpec((_R, _C), lambda i: (i, 0)),
        pl.BlockSpec((_R, _C), lambda i: (i, 0)),
        pl.BlockSpec((_R, _C), lambda i: (i, 0)),
        pl.BlockSpec((_R, 1), lambda i: (i, 0)),
        pl.BlockSpec((1, _C), lambda i: (0, 0)),
    ],
    out_specs=[
        pl.BlockSpec((_R, _C), lambda i: (i, 0)),
        pl.BlockSpec((_R, _C), lambda i: (i, 0)),
    ],
    out_shape=[
        jax.ShapeDtypeStruct((_N, _C), jnp.float32),
        jax.ShapeDtypeStruct((_N, _C), jnp.float32),
    ],
)


def _ortho_body(w1_ref, w2_ref, o_ref):
    w1 = w1_ref[...]
    w2 = w2_ref[...]
    g1 = lax.dot_general(w1, w1, (((1,), (1,)), ((), ())),
                         preferred_element_type=jnp.float32)
    g2 = lax.dot_general(w2, w2, (((1,), (1,)), ((), ())),
                         preferred_element_type=jnp.float32)
    i1 = (lax.broadcasted_iota(jnp.int32, (_F_IN, _F_IN), 0)
          == lax.broadcasted_iota(jnp.int32, (_F_IN, _F_IN), 1)).astype(jnp.float32)
    i2 = (lax.broadcasted_iota(jnp.int32, (_HID, _HID), 0)
          == lax.broadcasted_iota(jnp.int32, (_HID, _HID), 1)).astype(jnp.float32)
    s1 = jnp.sum((g1 - i1) ** 2)
    s2 = jnp.sum((g2 - i2) ** 2)
    o_ref[...] = jnp.reshape(jnp.sqrt(s1) + jnp.sqrt(s2), (1, 1))


_ortho = pl.pallas_call(
    _ortho_body,
    out_shape=jax.ShapeDtypeStruct((1, 1), jnp.float32),
)


def kernel(x, edge_index, W1, b1, W2, b2):
    row = edge_index[0].reshape(_ROWS, _CHUNK)
    col = edge_index[1].reshape(_ROWS, _CHUNK)
    z1 = jnp.zeros((_NPAD,), jnp.float32)
    z64 = jnp.zeros((_NPAD, _HID), jnp.float32)
    z16 = jnp.zeros((_NPAD, _C), jnp.float32)

    cnt0, cnt1 = _sc_degree(col, z1)
    y1, dinv = _tc1(x, W1, cnt0.reshape(_NPAD, 1), cnt1.reshape(_NPAD, 1))
    p0, p1 = _sc_scatter_hid(row, col, y1, z64)
    (z2,) = _tc2(p0, p1, y1, dinv, b1.reshape(1, _HID), W2)
    q0, q1 = _sc_scatter_out(row, col, z2, z16)
    logp, xout = _tc3(q0, q1, z2, dinv, b2.reshape(1, _C))
    orto = _ortho(W1, W2)
    return (logp, xout, orto.reshape(()))


# speedup vs baseline: 42.9862x; 1.3482x over previous
import functools

import jax
import jax.numpy as jnp
from jax import lax
from jax.experimental import pallas as pl
from jax.experimental.pallas import tpu as pltpu
from jax.experimental.pallas import tpu_sc as plsc

_N = 10000
_E = 320000
_F_IN = 128
_HID = 64
_C = 16

_NC = 2
_NS = 16
_NW = _NC * _NS
_CHUNK = 125
_ROWS = _E // _CHUNK
_ROWS_W = _ROWS // _NW
_NPAD = 10240
_NPS = _NPAD // _NS

_mesh = plsc.VectorSubcoreMesh(
    core_axis_name="c", subcore_axis_name="s", num_cores=_NC, num_subcores=_NS
)


@functools.partial(
    pl.kernel,
    out_type=[
        jax.ShapeDtypeStruct((_NPAD,), jnp.float32),
        jax.ShapeDtypeStruct((_NPAD,), jnp.float32),
    ],
    mesh=_mesh,
    compiler_params=pltpu.CompilerParams(use_tc_tiling_on_sc=False),
    scratch_types=[
        pltpu.VMEM((_ROWS_W, _CHUNK), jnp.int32),
        pltpu.VMEM((128,), jnp.float32),
        pltpu.VMEM_SHARED((_NPAD,), jnp.float32),
    ],
)
def _sc_degree(col_hbm, z1_hbm, cnt0_hbm, cnt1_hbm, colv, ones_v, acc):
    cid = lax.axis_index("c")
    sid = lax.axis_index("s")
    wid = sid * _NC + cid
    pltpu.sync_copy(z1_hbm.at[pl.ds(sid * _NPS, _NPS)], acc.at[pl.ds(sid * _NPS, _NPS)])
    pltpu.sync_copy(col_hbm.at[pl.ds(wid * _ROWS_W, _ROWS_W)], colv)
    for k in range(8):
        ones_v[pl.ds(k * 16, 16)] = jnp.ones((16,), jnp.float32)
    plsc.subcore_barrier()

    def body(j, carry):
        pltpu.sync_copy(ones_v.at[pl.ds(0, _CHUNK)], acc.at[colv.at[j]], add=True)
        return carry

    lax.fori_loop(0, _ROWS_W, body, 0)
    plsc.subcore_barrier()

    @pl.when(cid == 0)
    def _():
        pltpu.sync_copy(acc.at[pl.ds(sid * _NPS, _NPS)], cnt0_hbm.at[pl.ds(sid * _NPS, _NPS)])

    @pl.when(cid == 1)
    def _():
        pltpu.sync_copy(acc.at[pl.ds(sid * _NPS, _NPS)], cnt1_hbm.at[pl.ds(sid * _NPS, _NPS)])


def _make_sc_scatter(depth):
    @functools.partial(
        pl.kernel,
        out_type=[
            jax.ShapeDtypeStruct((_NPAD, depth), jnp.float32),
            jax.ShapeDtypeStruct((_NPAD, depth), jnp.float32),
        ],
        mesh=_mesh,
        compiler_params=pltpu.CompilerParams(use_tc_tiling_on_sc=False),
        scratch_types=[
            pltpu.VMEM((_ROWS_W, _CHUNK), jnp.int32),
            pltpu.VMEM((_ROWS_W, _CHUNK), jnp.int32),
            pltpu.VMEM((_CHUNK, depth), jnp.float32),
            pltpu.VMEM((_CHUNK, depth), jnp.float32),
            pltpu.VMEM_SHARED((_NPAD, depth), jnp.float32),
            pltpu.SemaphoreType.DMA,
            pltpu.SemaphoreType.DMA,
        ],
    )
    def _sc_scatter(row_hbm, col_hbm, y_hbm, zd_hbm, p0_hbm, p1_hbm,
                    rowv, colv, buf_a, buf_b, acc, sem_a, sem_b):
        cid = lax.axis_index("c")
        sid = lax.axis_index("s")
        wid = sid * _NC + cid
        pltpu.sync_copy(zd_hbm.at[pl.ds(sid * _NPS, _NPS)],
                        acc.at[pl.ds(sid * _NPS, _NPS)])
        pltpu.sync_copy(row_hbm.at[pl.ds(wid * _ROWS_W, _ROWS_W)], rowv)
        pltpu.sync_copy(col_hbm.at[pl.ds(wid * _ROWS_W, _ROWS_W)], colv)
        plsc.subcore_barrier()

        pltpu.async_copy(y_hbm.at[rowv.at[0]], buf_a, sem_a)

        def body(i, carry):
            j0 = 2 * i
            pltpu.async_copy(y_hbm.at[rowv.at[j0 + 1]], buf_b, sem_b)
            pltpu.make_async_copy(y_hbm.at[rowv.at[j0]], buf_a, sem_a).wait()
            pltpu.sync_copy(buf_a, acc.at[colv.at[j0]], add=True)

            @pl.when(j0 + 2 < _ROWS_W)
            def _():
                pltpu.async_copy(y_hbm.at[rowv.at[j0 + 2]], buf_a, sem_a)

            pltpu.make_async_copy(y_hbm.at[rowv.at[j0 + 1]], buf_b, sem_b).wait()
            pltpu.sync_copy(buf_b, acc.at[colv.at[j0 + 1]], add=True)
            return carry

        lax.fori_loop(0, _ROWS_W // 2, body, 0)
        plsc.subcore_barrier()

        @pl.when(cid == 0)
        def _():
            pltpu.sync_copy(acc.at[pl.ds(sid * _NPS, _NPS)],
                            p0_hbm.at[pl.ds(sid * _NPS, _NPS)])

        @pl.when(cid == 1)
        def _():
            pltpu.sync_copy(acc.at[pl.ds(sid * _NPS, _NPS)],
                            p1_hbm.at[pl.ds(sid * _NPS, _NPS)])

    return _sc_scatter


_sc_scatter_hid = _make_sc_scatter(_HID)
_sc_scatter_out = _make_sc_scatter(_C)


_R = 1000
_G = _N // _R


def _tc1_body(x_ref, w1_ref, c0_ref, c1_ref, y_ref, dinv_ref):
    deg = c0_ref[...] + c1_ref[...] + 1.0
    dinv = lax.rsqrt(deg)
    xw = jnp.dot(x_ref[...], w1_ref[...], preferred_element_type=jnp.float32)
    y_ref[...] = xw * dinv
    dinv_ref[...] = dinv


_tc1 = pl.pallas_call(
    _tc1_body,
    grid=(_G,),
    in_specs=[
        pl.BlockSpec((_R, _F_IN), lambda i: (i, 0)),
        pl.BlockSpec((_F_IN, _HID), lambda i: (0, 0)),
        pl.BlockSpec((_R, 1), lambda i: (i, 0)),
        pl.BlockSpec((_R, 1), lambda i: (i, 0)),
    ],
    out_specs=[
        pl.BlockSpec((_R, _HID), lambda i: (i, 0)),
        pl.BlockSpec((_R, 1), lambda i: (i, 0)),
    ],
    out_shape=[
        jax.ShapeDtypeStruct((_N, _HID), jnp.float32),
        jax.ShapeDtypeStruct((_N, 1), jnp.float32),
    ],
)


def _tc2_body(p0_ref, p1_ref, y1_ref, dinv_ref, b1_ref, w2_ref, z_ref):
    dinv = dinv_ref[...]
    out1 = (p0_ref[...] + p1_ref[...] + y1_ref[...]) * dinv + b1_ref[...]
    h = jnp.maximum(out1, 0.0)
    z_ref[...] = jnp.dot(h, w2_ref[...], preferred_element_type=jnp.float32) * dinv


_tc2 = pl.pallas_call(
    _tc2_body,
    grid=(_G,),
    in_specs=[
        pl.BlockSpec((_R, _HID), lambda i: (i, 0)),
        pl.BlockSpec((_R, _HID), lambda i: (i, 0)),
        pl.BlockSpec((_R, _HID), lambda i: (i, 0)),
        pl.BlockSpec((_R, 1), lambda i: (i, 0)),
        pl.BlockSpec((1, _HID), lambda i: (0, 0)),
        pl.BlockSpec((_HID, _C), lambda i: (0, 0)),
    ],
    out_specs=[pl.BlockSpec((_R, _C), lambda i: (i, 0))],
    out_shape=[jax.ShapeDtypeStruct((_N, _C), jnp.float32)],
)


def _tc3_body(q0_ref, q1_ref, z2_ref, dinv_ref, b2_ref, logp_ref, xout_ref):
    xo = (q0_ref[...] + q1_ref[...] + z2_ref[...]) * dinv_ref[...] + b2_ref[...]
    m = jnp.max(xo, axis=1, keepdims=True)
    t = xo - m
    lse = jnp.log(jnp.sum(jnp.exp(t), axis=1, keepdims=True))
    logp_ref[...] = t - lse
    xout_ref[...] = xo


_tc3 = pl.pallas_call(
    _tc3_body,
    grid=(_G,),
    in_specs=[
        pl.BlockSpec((_R, _C), lambda i: (i, 0)),
        pl.BlockSpec((_R, _C), lambda i: (i, 0)),
        pl.BlockSpec((_R, _C), lambda i: (i, 0)),
        pl.BlockSpec((_R, 1), lambda i: (i, 0)),
        pl.BlockSpec((1, _C), lambda i: (0, 0)),
    ],
    out_specs=[
        pl.BlockSpec((_R, _C), lambda i: (i, 0)),
        pl.BlockSpec((_R, _C), lambda i: (i, 0)),
    ],
    out_shape=[
        jax.ShapeDtypeStruct((_N, _C), jnp.float32),
        jax.ShapeDtypeStruct((_N, _C), jnp.float32),
    ],
)


def _ortho_body(w1_ref, w2_ref, o_ref):
    w1 = w1_ref[...]
    w2 = w2_ref[...]
    g1 = lax.dot_general(w1, w1, (((1,), (1,)), ((), ())),
                         preferred_element_type=jnp.float32)
    g2 = lax.dot_general(w2, w2, (((1,), (1,)), ((), ())),
                         preferred_element_type=jnp.float32)
    i1 = (lax.broadcasted_iota(jnp.int32, (_F_IN, _F_IN), 0)
          == lax.broadcasted_iota(jnp.int32, (_F_IN, _F_IN), 1)).astype(jnp.float32)
    i2 = (lax.broadcasted_iota(jnp.int32, (_HID, _HID), 0)
          == lax.broadcasted_iota(jnp.int32, (_HID, _HID), 1)).astype(jnp.float32)
    s1 = jnp.sum((g1 - i1) ** 2)
    s2 = jnp.sum((g2 - i2) ** 2)
    o_ref[...] = jnp.reshape(jnp.sqrt(s1) + jnp.sqrt(s2), (1, 1))


_ortho = pl.pallas_call(
    _ortho_body,
    out_shape=jax.ShapeDtypeStruct((1, 1), jnp.float32),
)


def kernel(x, edge_index, W1, b1, W2, b2):
    row = edge_index[0].reshape(_ROWS, _CHUNK)
    col = edge_index[1].reshape(_ROWS, _CHUNK)
    z1 = jnp.zeros((_NPAD,), jnp.float32)
    z64 = jnp.zeros((_NPAD, _HID), jnp.float32)
    z16 = jnp.zeros((_NPAD, _C), jnp.float32)

    cnt0, cnt1 = _sc_degree(col, z1)
    y1, dinv = _tc1(x, W1, cnt0.reshape(_NPAD, 1), cnt1.reshape(_NPAD, 1))
    p0, p1 = _sc_scatter_hid(row, col, y1, z64)
    (z2,) = _tc2(p0, p1, y1, dinv, b1.reshape(1, _HID), W2)
    q0, q1 = _sc_scatter_out(row, col, z2, z16)
    logp, xout = _tc3(q0, q1, z2, dinv, b2.reshape(1, _C))
    orto = _ortho(W1, W2)
    return (logp, xout, orto.reshape(()))


# speedup vs baseline: 49.6799x; 1.1557x over previous
import functools

import jax
import jax.numpy as jnp
from jax import lax
from jax.experimental import pallas as pl
from jax.experimental.pallas import tpu as pltpu
from jax.experimental.pallas import tpu_sc as plsc

_N = 10000
_E = 320000
_F_IN = 128
_HID = 64
_C = 16

_NC = 2
_NS = 16
_NW = _NC * _NS
_CHUNK = 125
_ROWS = _E // _CHUNK
_ROWS_W = _ROWS // _NW
_NPAD = 10240
_NPS = _NPAD // _NS

_mesh = plsc.VectorSubcoreMesh(
    core_axis_name="c", subcore_axis_name="s", num_cores=_NC, num_subcores=_NS
)


@functools.partial(
    pl.kernel,
    out_type=[
        jax.ShapeDtypeStruct((_NPAD,), jnp.float32),
        jax.ShapeDtypeStruct((_NPAD,), jnp.float32),
    ],
    mesh=_mesh,
    compiler_params=pltpu.CompilerParams(use_tc_tiling_on_sc=False),
    scratch_types=[
        pltpu.VMEM((_ROWS_W, _CHUNK), jnp.int32),
        pltpu.VMEM((128,), jnp.float32),
        pltpu.VMEM_SHARED((_NPAD,), jnp.float32),
    ],
)
def _sc_degree(col_hbm, z1_hbm, cnt0_hbm, cnt1_hbm, colv, ones_v, acc):
    cid = lax.axis_index("c")
    sid = lax.axis_index("s")
    wid = sid * _NC + cid
    pltpu.sync_copy(z1_hbm.at[pl.ds(sid * _NPS, _NPS)], acc.at[pl.ds(sid * _NPS, _NPS)])
    pltpu.sync_copy(col_hbm.at[pl.ds(wid * _ROWS_W, _ROWS_W)], colv)
    for k in range(8):
        ones_v[pl.ds(k * 16, 16)] = jnp.ones((16,), jnp.float32)
    plsc.subcore_barrier()

    def body(j, carry):
        pltpu.sync_copy(ones_v.at[pl.ds(0, _CHUNK)], acc.at[colv.at[j]], add=True)
        return carry

    lax.fori_loop(0, _ROWS_W, body, 0)
    plsc.subcore_barrier()

    @pl.when(cid == 0)
    def _():
        pltpu.sync_copy(acc.at[pl.ds(sid * _NPS, _NPS)], cnt0_hbm.at[pl.ds(sid * _NPS, _NPS)])

    @pl.when(cid == 1)
    def _():
        pltpu.sync_copy(acc.at[pl.ds(sid * _NPS, _NPS)], cnt1_hbm.at[pl.ds(sid * _NPS, _NPS)])


def _make_sc_scatter(depth):
    @functools.partial(
        pl.kernel,
        out_type=[
            jax.ShapeDtypeStruct((_NPAD, depth), jnp.float32),
            jax.ShapeDtypeStruct((_NPAD, depth), jnp.float32),
        ],
        mesh=_mesh,
        compiler_params=pltpu.CompilerParams(use_tc_tiling_on_sc=False),
        scratch_types=[
            pltpu.VMEM((_ROWS_W, _CHUNK), jnp.int32),
            pltpu.VMEM((_ROWS_W, _CHUNK), jnp.int32),
            pltpu.VMEM((_CHUNK, depth), jnp.float32),
            pltpu.VMEM((_CHUNK, depth), jnp.float32),
            pltpu.VMEM((_CHUNK, depth), jnp.float32),
            pltpu.VMEM((_CHUNK, depth), jnp.float32),
            pltpu.VMEM_SHARED((_NPAD, depth), jnp.float32),
            pltpu.SemaphoreType.DMA,
            pltpu.SemaphoreType.DMA,
            pltpu.SemaphoreType.DMA,
            pltpu.SemaphoreType.DMA,
        ],
    )
    def _sc_scatter(row_hbm, col_hbm, y_hbm, zd_hbm, p0_hbm, p1_hbm,
                    rowv, colv, buf_a, buf_b, buf_c, buf_d, acc,
                    sem_a, sem_b, sem_c, sem_d):
        cid = lax.axis_index("c")
        sid = lax.axis_index("s")
        wid = sid * _NC + cid
        bufs = (buf_a, buf_b, buf_c, buf_d)
        sems = (sem_a, sem_b, sem_c, sem_d)
        nbuf = 4
        pltpu.sync_copy(zd_hbm.at[pl.ds(sid * _NPS, _NPS)],
                        acc.at[pl.ds(sid * _NPS, _NPS)])
        pltpu.sync_copy(row_hbm.at[pl.ds(wid * _ROWS_W, _ROWS_W)], rowv)
        pltpu.sync_copy(col_hbm.at[pl.ds(wid * _ROWS_W, _ROWS_W)], colv)
        plsc.subcore_barrier()

        for b in range(nbuf):
            pltpu.async_copy(y_hbm.at[rowv.at[b]], bufs[b], sems[b])

        def body(i, carry):
            for b in range(nbuf):
                j = nbuf * i + b
                pltpu.make_async_copy(y_hbm.at[rowv.at[j]], bufs[b], sems[b]).wait()
                pltpu.sync_copy(bufs[b], acc.at[colv.at[j]], add=True)

                @pl.when(j + nbuf < _ROWS_W)
                def _():
                    pltpu.async_copy(y_hbm.at[rowv.at[j + nbuf]], bufs[b], sems[b])

            return carry

        lax.fori_loop(0, _ROWS_W // nbuf, body, 0)
        plsc.subcore_barrier()

        @pl.when(cid == 0)
        def _():
            pltpu.sync_copy(acc.at[pl.ds(sid * _NPS, _NPS)],
                            p0_hbm.at[pl.ds(sid * _NPS, _NPS)])

        @pl.when(cid == 1)
        def _():
            pltpu.sync_copy(acc.at[pl.ds(sid * _NPS, _NPS)],
                            p1_hbm.at[pl.ds(sid * _NPS, _NPS)])

    return _sc_scatter


_sc_scatter_hid = _make_sc_scatter(_HID)
_sc_scatter_out = _make_sc_scatter(_C)


_R = 1000
_G = _N // _R


def _tc1_body(x_ref, w1_ref, c0_ref, c1_ref, y_ref, dinv_ref):
    deg = c0_ref[...] + c1_ref[...] + 1.0
    dinv = lax.rsqrt(deg)
    xw = jnp.dot(x_ref[...], w1_ref[...], preferred_element_type=jnp.float32)
    y_ref[...] = xw * dinv
    dinv_ref[...] = dinv


_tc1 = pl.pallas_call(
    _tc1_body,
    grid=(_G,),
    in_specs=[
        pl.BlockSpec((_R, _F_IN), lambda i: (i, 0)),
        pl.BlockSpec((_F_IN, _HID), lambda i: (0, 0)),
        pl.BlockSpec((_R, 1), lambda i: (i, 0)),
        pl.BlockSpec((_R, 1), lambda i: (i, 0)),
    ],
    out_specs=[
        pl.BlockSpec((_R, _HID), lambda i: (i, 0)),
        pl.BlockSpec((_R, 1), lambda i: (i, 0)),
    ],
    out_shape=[
        jax.ShapeDtypeStruct((_N, _HID), jnp.float32),
        jax.ShapeDtypeStruct((_N, 1), jnp.float32),
    ],
)


def _tc2_body(p0_ref, p1_ref, y1_ref, dinv_ref, b1_ref, w2_ref, z_ref):
    dinv = dinv_ref[...]
    out1 = (p0_ref[...] + p1_ref[...] + y1_ref[...]) * dinv + b1_ref[...]
    h = jnp.maximum(out1, 0.0)
    z_ref[...] = jnp.dot(h, w2_ref[...], preferred_element_type=jnp.float32) * dinv


_tc2 = pl.pallas_call(
    _tc2_body,
    grid=(_G,),
    in_specs=[
        pl.BlockSpec((_R, _HID), lambda i: (i, 0)),
        pl.BlockSpec((_R, _HID), lambda i: (i, 0)),
        pl.BlockSpec((_R, _HID), lambda i: (i, 0)),
        pl.BlockSpec((_R, 1), lambda i: (i, 0)),
        pl.BlockSpec((1, _HID), lambda i: (0, 0)),
        pl.BlockSpec((_HID, _C), lambda i: (0, 0)),
    ],
    out_specs=[pl.BlockSpec((_R, _C), lambda i: (i, 0))],
    out_shape=[jax.ShapeDtypeStruct((_N, _C), jnp.float32)],
)


def _tc3_body(q0_ref, q1_ref, z2_ref, dinv_ref, b2_ref, logp_ref, xout_ref):
    xo = (q0_ref[...] + q1_ref[...] + z2_ref[...]) * dinv_ref[...] + b2_ref[...]
    m = jnp.max(xo, axis=1, keepdims=True)
    t = xo - m
    lse = jnp.log(jnp.sum(jnp.exp(t), axis=1, keepdims=True))
    logp_ref[...] = t - lse
    xout_ref[...] = xo


_tc3 = pl.pallas_call(
    _tc3_body,
    grid=(_G,),
    in_specs=[
        pl.BlockSpec((_R, _C), lambda i: (i, 0)),
        pl.BlockSpec((_R, _C), lambda i: (i, 0)),
        pl.BlockSpec((_R, _C), lambda i: (i, 0)),
        pl.BlockSpec((_R, 1), lambda i: (i, 0)),
        pl.BlockSpec((1, _C), lambda i: (0, 0)),
    ],
    out_specs=[
        pl.BlockSpec((_R, _C), lambda i: (i, 0)),
        pl.BlockSpec((_R, _C), lambda i: (i, 0)),
    ],
    out_shape=[
        jax.ShapeDtypeStruct((_N, _C), jnp.float32),
        jax.ShapeDtypeStruct((_N, _C), jnp.float32),
    ],
)


def _ortho_body(w1_ref, w2_ref, o_ref):
    w1 = w1_ref[...]
    w2 = w2_ref[...]
    g1 = lax.dot_general(w1, w1, (((1,), (1,)), ((), ())),
                         preferred_element_type=jnp.float32)
    g2 = lax.dot_general(w2, w2, (((1,), (1,)), ((), ())),
                         preferred_element_type=jnp.float32)
    i1 = (lax.broadcasted_iota(jnp.int32, (_F_IN, _F_IN), 0)
          == lax.broadcasted_iota(jnp.int32, (_F_IN, _F_IN), 1)).astype(jnp.float32)
    i2 = (lax.broadcasted_iota(jnp.int32, (_HID, _HID), 0)
          == lax.broadcasted_iota(jnp.int32, (_HID, _HID), 1)).astype(jnp.float32)
    s1 = jnp.sum((g1 - i1) ** 2)
    s2 = jnp.sum((g2 - i2) ** 2)
    o_ref[...] = jnp.reshape(jnp.sqrt(s1) + jnp.sqrt(s2), (1, 1))


_ortho = pl.pallas_call(
    _ortho_body,
    out_shape=jax.ShapeDtypeStruct((1, 1), jnp.float32),
)


def kernel(x, edge_index, W1, b1, W2, b2):
    row = edge_index[0].reshape(_ROWS, _CHUNK)
    col = edge_index[1].reshape(_ROWS, _CHUNK)
    z1 = jnp.zeros((_NPAD,), jnp.float32)
    z64 = jnp.zeros((_NPAD, _HID), jnp.float32)
    z16 = jnp.zeros((_NPAD, _C), jnp.float32)

    cnt0, cnt1 = _sc_degree(col, z1)
    y1, dinv = _tc1(x, W1, cnt0.reshape(_NPAD, 1), cnt1.reshape(_NPAD, 1))
    p0, p1 = _sc_scatter_hid(row, col, y1, z64)
    (z2,) = _tc2(p0, p1, y1, dinv, b1.reshape(1, _HID), W2)
    q0, q1 = _sc_scatter_out(row, col, z2, z16)
    logp, xout = _tc3(q0, q1, z2, dinv, b2.reshape(1, _C))
    orto = _ortho(W1, W2)
    return (logp, xout, orto.reshape(()))
